# baseline (speedup 1.0000x reference)
import functools

import jax
import jax.numpy as jnp
from jax import lax
from jax.experimental import pallas as pl
from jax.experimental.pallas import tpu as pltpu
from jax.experimental.pallas import tpu_sc as plsc

EMBED = 128
N_PLAYERS = 6
N_TYPES = 3
BATCH = 16384
ROWS = BATCH * N_PLAYERS
NC, NS = 2, 16
NW = NC * NS
R_PER_W = ROWS // NW
CH = 192
NCH = R_PER_W // CH
CHE = CH * EMBED


@functools.cache
def _build_sc_add():
  mesh = plsc.VectorSubcoreMesh(core_axis_name="c", subcore_axis_name="s")

  @functools.partial(
      pl.kernel,
      out_type=jax.ShapeDtypeStruct((ROWS * EMBED,), jnp.float32),
      mesh=mesh,
      compiler_params=pltpu.CompilerParams(needs_layout_passes=False),
      scratch_types=[
          pltpu.VMEM((CHE,), jnp.float32),
          pltpu.VMEM((CHE,), jnp.float32),
          pltpu.VMEM((CHE,), jnp.float32),
          pltpu.VMEM((CHE,), jnp.float32),
          pltpu.VMEM((CH,), jnp.int32),
          pltpu.VMEM((CH,), jnp.int32),
          pltpu.VMEM((N_PLAYERS * EMBED,), jnp.float32),
          pltpu.VMEM((N_TYPES * EMBED,), jnp.float32),
          pltpu.VMEM((N_PLAYERS * N_TYPES * EMBED,), jnp.float32),
          pltpu.SemaphoreType.DMA((2,)),
          pltpu.SemaphoreType.DMA((2,)),
          pltpu.SemaphoreType.DMA((2,)),
      ],
  )
  def _sc_add(x_hbm, t_hbm, te_hbm, pe_hbm, out_hbm,
              xbuf0, xbuf1, obuf0, obuf1, tbuf0, tbuf1,
              pe_v, te_v, cbuf,
              xin_sem, tin_sem, out_sem):
    xbufs = (xbuf0, xbuf1)
    obufs = (obuf0, obuf1)
    tbufs = (tbuf0, tbuf1)
    wid = lax.axis_index("s") * NC + lax.axis_index("c")
    base = wid * R_PER_W

    pltpu.sync_copy(pe_hbm, pe_v)
    pltpu.sync_copy(te_hbm, te_v)
    for p in range(N_PLAYERS):
      for t in range(N_TYPES):
        for j in range(EMBED // 16):
          cbuf[pl.ds((p * N_TYPES + t) * EMBED + j * 16, 16)] = (
              pe_v[pl.ds(p * EMBED + j * 16, 16)]
              + te_v[pl.ds(t * EMBED + j * 16, 16)])

    def start_in(g, b):
      pltpu.async_copy(x_hbm.at[pl.ds((base + g * CH) * EMBED, CHE)],
                       xbufs[b], xin_sem.at[b])
      pltpu.async_copy(t_hbm.at[pl.ds(base + g * CH, CH)],
                       tbufs[b], tin_sem.at[b])

    def wait_in(b):
      pltpu.make_async_copy(x_hbm.at[pl.ds(0, CHE)], xbufs[b],
                            xin_sem.at[b]).wait()
      pltpu.make_async_copy(t_hbm.at[pl.ds(0, CH)], tbufs[b],
                            tin_sem.at[b]).wait()

    def start_out(g, b):
      pltpu.async_copy(obufs[b],
                       out_hbm.at[pl.ds((base + g * CH) * EMBED, CHE)],
                       out_sem.at[b])

    def wait_out(b):
      pltpu.make_async_copy(obufs[b], out_hbm.at[pl.ds(0, CHE)],
                            out_sem.at[b]).wait()

    lanes = lax.iota(jnp.int32, 16)

    def compute(g, b):
      chunk_base = base + g * CH
      for g16 in range(CH // 16):
        rows = g16 * 16 + lanes
        t = tbufs[b][pl.ds(g16 * 16, 16)]
        p = lax.rem(chunk_base + rows, N_PLAYERS)
        cidx = p * N_TYPES + t
        xi0 = rows * EMBED
        ci0 = cidx * EMBED

        def col_body(col, _):
          xi = xi0 + col
          xv = plsc.load_gather(xbufs[b], [xi])
          cv = plsc.load_gather(cbuf, [ci0 + col])
          plsc.store_scatter(obufs[b], [xi], xv + cv)
          return 0

        lax.fori_loop(0, EMBED, col_body, 0)

    start_in(0, 0)
    start_in(1, 1)
    for g in range(NCH):
      b = g % 2
      wait_in(b)
      if g >= 2:
        wait_out(b)
      compute(g, b)
      start_out(g, b)
      if g + 2 < NCH:
        start_in(g + 2, b)
    wait_out(0)
    wait_out(1)

  return _sc_add


def kernel(x, entity_types, entity_type_embedding, position_embedding):
  x_flat = x.reshape(ROWS * EMBED)
  t_flat = entity_types.reshape(ROWS).astype(jnp.int32)
  out = _build_sc_add()(x_flat, t_flat, entity_type_embedding.reshape(-1),
                        position_embedding.reshape(-1))
  return out.reshape(x.shape)


# speedup vs baseline: 1.0015x; 1.0015x over previous
import functools

import jax
import jax.numpy as jnp
from jax import lax
from jax.experimental import pallas as pl
from jax.experimental.pallas import tpu as pltpu
from jax.experimental.pallas import tpu_sc as plsc

EMBED = 128
N_PLAYERS = 6
N_TYPES = 3
BATCH = 16384
ROWS = BATCH * N_PLAYERS
NC, NS = 2, 16
NW = NC * NS
R_PER_W = ROWS // NW
CH = 192
NCH = R_PER_W // CH
CHE = CH * EMBED


@functools.cache
def _build_sc_add():
  mesh = plsc.VectorSubcoreMesh(core_axis_name="c", subcore_axis_name="s")

  @functools.partial(
      pl.kernel,
      out_type=jax.ShapeDtypeStruct((ROWS * EMBED,), jnp.float32),
      mesh=mesh,
      compiler_params=pltpu.CompilerParams(needs_layout_passes=False),
      scratch_types=[
          pltpu.VMEM((CHE,), jnp.float32),
          pltpu.VMEM((CHE,), jnp.float32),
          pltpu.VMEM((CHE,), jnp.float32),
          pltpu.VMEM((CHE,), jnp.float32),
          pltpu.VMEM((CH,), jnp.int32),
          pltpu.VMEM((CH,), jnp.int32),
          pltpu.VMEM((N_PLAYERS * EMBED,), jnp.float32),
          pltpu.VMEM((N_TYPES * EMBED,), jnp.float32),
          pltpu.VMEM((N_PLAYERS * N_TYPES * EMBED,), jnp.float32),
          pltpu.SemaphoreType.DMA((2,)),
          pltpu.SemaphoreType.DMA((2,)),
          pltpu.SemaphoreType.DMA((2,)),
      ],
  )
  def _sc_add(x_hbm, t_hbm, te_hbm, pe_hbm, out_hbm,
              xbuf0, xbuf1, obuf0, obuf1, tbuf0, tbuf1,
              pe_v, te_v, cbuf,
              xin_sem, tin_sem, out_sem):
    xbufs = (xbuf0, xbuf1)
    obufs = (obuf0, obuf1)
    tbufs = (tbuf0, tbuf1)
    wid = lax.axis_index("s") * NC + lax.axis_index("c")
    base = wid * R_PER_W

    pltpu.sync_copy(pe_hbm, pe_v)
    pltpu.sync_copy(te_hbm, te_v)
    for p in range(N_PLAYERS):
      for t in range(N_TYPES):
        for j in range(EMBED // 16):
          cbuf[pl.ds((p * N_TYPES + t) * EMBED + j * 16, 16)] = (
              pe_v[pl.ds(p * EMBED + j * 16, 16)]
              + te_v[pl.ds(t * EMBED + j * 16, 16)])

    def start_in(g, b):
      pltpu.async_copy(x_hbm.at[pl.ds((base + g * CH) * EMBED, CHE)],
                       xbufs[b], xin_sem.at[b])
      pltpu.async_copy(t_hbm.at[pl.ds(base + g * CH, CH)],
                       tbufs[b], tin_sem.at[b])

    def wait_in(b):
      pltpu.make_async_copy(x_hbm.at[pl.ds(0, CHE)], xbufs[b],
                            xin_sem.at[b]).wait()
      pltpu.make_async_copy(t_hbm.at[pl.ds(0, CH)], tbufs[b],
                            tin_sem.at[b]).wait()

    def start_out(g, b):
      pltpu.async_copy(obufs[b],
                       out_hbm.at[pl.ds((base + g * CH) * EMBED, CHE)],
                       out_sem.at[b])

    def wait_out(b):
      pltpu.make_async_copy(obufs[b], out_hbm.at[pl.ds(0, CHE)],
                            out_sem.at[b]).wait()

    lanes = lax.iota(jnp.int32, 16)

    def compute(g, b):
      chunk_base = base + g * CH

      def g16_body(i, _):
        off = i * 16
        rows = off + lanes
        t = tbufs[b][pl.ds(off, 16)]
        p = lax.rem(chunk_base + rows, N_PLAYERS)
        cidx = p * N_TYPES + t
        xi0 = rows * EMBED
        ci0 = cidx * EMBED

        def blk_body(jb, _):
          xib = xi0 + jb * 16
          cib = ci0 + jb * 16
          for k in range(16):
            xi = xib + k
            xv = plsc.load_gather(xbufs[b], [xi])
            cv = plsc.load_gather(cbuf, [cib + k])
            plsc.store_scatter(obufs[b], [xi], xv + cv)
          return 0

        lax.fori_loop(0, EMBED // 16, blk_body, 0)
        return 0

      lax.fori_loop(0, CH // 16, g16_body, 0)

    start_in(0, 0)
    start_in(1, 1)
    for g in range(NCH):
      b = g % 2
      wait_in(b)
      if g >= 2:
        wait_out(b)
      compute(g, b)
      start_out(g, b)
      if g + 2 < NCH:
        start_in(g + 2, b)
    wait_out(0)
    wait_out(1)

  return _sc_add


def kernel(x, entity_types, entity_type_embedding, position_embedding):
  x_flat = x.reshape(ROWS * EMBED)
  t_flat = entity_types.reshape(ROWS).astype(jnp.int32)
  out = _build_sc_add()(x_flat, t_flat, entity_type_embedding.reshape(-1),
                        position_embedding.reshape(-1))
  return out.reshape(x.shape)


# speedup vs baseline: 2.5569x; 2.5531x over previous
import functools

import jax
import jax.numpy as jnp
from jax import lax
from jax.experimental import pallas as pl
from jax.experimental.pallas import tpu as pltpu
from jax.experimental.pallas import tpu_sc as plsc

EMBED = 128
N_PLAYERS = 6
N_TYPES = 3
BATCH = 16384
ROWS = BATCH * N_PLAYERS
NC, NS = 2, 16
NW = NC * NS
R_PER_W = ROWS // NW
CH = 192
NCH = R_PER_W // CH
CHE = CH * EMBED


@functools.cache
def _build_sc_add():
  mesh = plsc.VectorSubcoreMesh(core_axis_name="c", subcore_axis_name="s")

  @functools.partial(
      pl.kernel,
      out_type=jax.ShapeDtypeStruct((ROWS * EMBED,), jnp.float32),
      mesh=mesh,
      compiler_params=pltpu.CompilerParams(needs_layout_passes=False),
      scratch_types=[
          pltpu.VMEM((CHE,), jnp.float32),
          pltpu.VMEM((CHE,), jnp.float32),
          pltpu.VMEM((CHE,), jnp.float32),
          pltpu.VMEM((CHE,), jnp.float32),
          pltpu.VMEM((CH,), jnp.int32),
          pltpu.VMEM((CH,), jnp.int32),
          pltpu.VMEM((N_PLAYERS * EMBED,), jnp.float32),
          pltpu.VMEM((N_TYPES * EMBED,), jnp.float32),
          pltpu.VMEM((N_PLAYERS * N_TYPES * EMBED,), jnp.float32),
          pltpu.SemaphoreType.DMA((2,)),
          pltpu.SemaphoreType.DMA((2,)),
          pltpu.SemaphoreType.DMA((2,)),
      ],
  )
  def _sc_add(x_hbm, t_hbm, te_hbm, pe_hbm, out_hbm,
              xbuf0, xbuf1, obuf0, obuf1, tbuf0, tbuf1,
              pe_v, te_v, cbuf,
              xin_sem, tin_sem, out_sem):
    xbufs = (xbuf0, xbuf1)
    obufs = (obuf0, obuf1)
    tbufs = (tbuf0, tbuf1)
    wid = lax.axis_index("s") * NC + lax.axis_index("c")
    base = wid * R_PER_W

    pltpu.sync_copy(pe_hbm, pe_v)
    pltpu.sync_copy(te_hbm, te_v)
    for p in range(N_PLAYERS):
      for t in range(N_TYPES):
        for j in range(EMBED // 16):
          cbuf[pl.ds((p * N_TYPES + t) * EMBED + j * 16, 16)] = (
              pe_v[pl.ds(p * EMBED + j * 16, 16)]
              + te_v[pl.ds(t * EMBED + j * 16, 16)])

    def start_in(g, b):
      pltpu.async_copy(x_hbm.at[pl.ds((base + g * CH) * EMBED, CHE)],
                       xbufs[b], xin_sem.at[b])
      pltpu.async_copy(t_hbm.at[pl.ds(base + g * CH, CH)],
                       tbufs[b], tin_sem.at[b])

    def wait_in(b):
      pltpu.make_async_copy(x_hbm.at[pl.ds(0, CHE)], xbufs[b],
                            xin_sem.at[b]).wait()
      pltpu.make_async_copy(t_hbm.at[pl.ds(0, CH)], tbufs[b],
                            tin_sem.at[b]).wait()

    def start_out(g, b):
      pltpu.async_copy(obufs[b],
                       out_hbm.at[pl.ds((base + g * CH) * EMBED, CHE)],
                       out_sem.at[b])

    def wait_out(b):
      pltpu.make_async_copy(obufs[b], out_hbm.at[pl.ds(0, CHE)],
                            out_sem.at[b]).wait()

    lanes = lax.iota(jnp.int32, 16)

    def compute(g, b):
      chunk_base = base + g * CH

      def g16_body(i, _):
        off = i * 16
        rows = off + lanes
        t = tbufs[b][pl.ds(off, 16)]
        p = lax.rem(chunk_base + rows, N_PLAYERS)
        ci0 = (p * N_TYPES + t) * EMBED

        def blk_body(jb, _):
          colv = jb * 16 + lanes
          for l in range(16):
            ci = ci0.at[jnp.full((16,), l, jnp.int32)].get(
                mode='promise_in_bounds')
            cv = plsc.load_gather(cbuf, [ci + colv])
            row_off = (off + l) * EMBED + jb * 16
            xv = xbufs[b][pl.ds(row_off, 16)]
            obufs[b][pl.ds(row_off, 16)] = xv + cv
          return 0

        lax.fori_loop(0, EMBED // 16, blk_body, 0)
        return 0

      lax.fori_loop(0, CH // 16, g16_body, 0)

    start_in(0, 0)
    start_in(1, 1)
    for g in range(NCH):
      b = g % 2
      wait_in(b)
      if g >= 2:
        wait_out(b)
      compute(g, b)
      start_out(g, b)
      if g + 2 < NCH:
        start_in(g + 2, b)
    wait_out(0)
    wait_out(1)

  return _sc_add


def kernel(x, entity_types, entity_type_embedding, position_embedding):
  x_flat = x.reshape(ROWS * EMBED)
  t_flat = entity_types.reshape(ROWS).astype(jnp.int32)
  out = _build_sc_add()(x_flat, t_flat, entity_type_embedding.reshape(-1),
                        position_embedding.reshape(-1))
  return out.reshape(x.shape)
